# baseline (speedup 1.0000x reference)
import functools

import jax
import jax.numpy as jnp
from jax import lax
from jax.experimental import pallas as pl
from jax.experimental.pallas import tpu as pltpu
from jax.experimental.pallas import tpu_sc as plsc

VOCAB = 30522
D = 768
B = 16
L = 512

NC = 2
NS = 16
NW = NC * NS
TOK = B * L
CH = TOK // NW
PW = L // CH
P = 64
NP = CH // P
NBUF = 2
G = 16
DV = D // 16


def _body(ids_hbm, len_hbm, table_hbm, out_hbm,
          idx2, slv, buf, zbuf, isem, zsem, gsem, ssem):
    wid = lax.axis_index("s") * NC + lax.axis_index("c")
    b = wid // PW
    l_start = (wid % PW) * CH

    def stage_idx(i):
        return pltpu.make_async_copy(
            ids_hbm.at[b, pl.ds(l_start + i * P, P)], idx2.at[i], isem)

    def gather_cp(j, s):
        return pltpu.make_async_copy(
            table_hbm.at[idx2.at[j]],
            buf.at[pl.ds(pl.multiple_of(s * P, P), P)], gsem.at[s])

    def unit_cp(j, s, i, sem):
        return pltpu.make_async_copy(
            buf.at[pl.ds(pl.multiple_of(s * P + i * G, G), G)],
            out_hbm.at[b, pl.ds(pl.multiple_of(l_start + j * P + i * G, G), G)],
            sem)

    stage_idx(0).start()
    stage_idx(1).start()
    stage_idx(0).wait()
    gather_cp(0, 0).start()
    stage_idx(1).wait()
    gather_cp(1, 1).start()
    for i in range(2, NP):
        stage_idx(i).start()

    pltpu.sync_copy(len_hbm, slv)
    for i in range(2, NP):
        stage_idx(i).wait()

    lane = lax.broadcasted_iota(jnp.int32, (16,), 0)
    sl = jnp.max(jnp.where(lane == b, slv[...], 0))
    nv = lax.max(lax.min(sl - l_start, CH), 0)

    zeros16 = jnp.zeros((16,), jnp.float32)

    def zrow(k, _):
        zbuf[k // DV, pl.ds((k % DV) * 16, 16)] = zeros16
        return 0

    lax.fori_loop(0, G * DV, zrow, 0)

    def valid_rows(j):
        return lax.max(lax.min(nv - j * P, P), 0)

    def step(j, ztot):
        s = j % NBUF
        row0 = l_start + j * P
        lo = valid_rows(j)
        a16 = ((lo + (G - 1)) // G) * G
        nu = a16 // G
        nz = (P - a16) // G

        def zfill(i, t):
            pltpu.make_async_copy(
                zbuf,
                out_hbm.at[b, pl.ds(pl.multiple_of(row0 + a16 + i * G, G), G)],
                zsem).start()
            return t + 1

        ztot = lax.fori_loop(0, nz, zfill, ztot)

        def gw(i, _):
            gather_cp(j, s).wait()
            return 0

        lax.fori_loop(0, jnp.where(j < 2, 1, lax.min(nu, 1)), gw, 0)

        def zo(k, _):
            buf[s * P + lo + k // DV, pl.ds((k % DV) * 16, 16)] = zeros16
            return 0

        lax.fori_loop(0, (a16 - lo) * DV, zo, 0)

        def wb(i, _):
            unit_cp(j, s, i, ssem.at[s]).start()
            return 0

        lax.fori_loop(0, nu, wb, 0)

        jj = j + 2
        s2 = s
        live = jj < NP
        drain_cnt = jnp.where(live, nu, 0)

        def sw(i, _):
            unit_cp(j, s, i, ssem.at[s]).wait()
            return 0

        lax.fori_loop(0, drain_cnt, sw, 0)

        loj = valid_rows(jj)
        fire_cnt = jnp.where(live, lax.min((loj + (G - 1)) // G, 1), 0)

        def gf(i, _):
            gather_cp(jj, s2).start()
            return 0

        lax.fori_loop(0, fire_cnt, gf, 0)
        return ztot

    ztot = lax.fori_loop(0, NP, step, jnp.int32(0))

    def ep(t, _):
        j = (NP - NBUF) + t
        s = j % NBUF
        nu = (valid_rows(j) + (G - 1)) // G

        def sw2(i, _):
            unit_cp(j, s, i, ssem.at[s]).wait()
            return 0

        lax.fori_loop(0, nu, sw2, 0)
        return 0

    lax.fori_loop(0, NBUF, ep, 0)

    def zdrain(i, _):
        pltpu.make_async_copy(
            zbuf, out_hbm.at[b, pl.ds(l_start, G)], zsem).wait()
        return 0

    lax.fori_loop(0, ztot, zdrain, 0)


@functools.partial(jax.jit, static_argnames=())
def kernel(seq_ids, seq_len, table):
    def body(ids_hbm, len_hbm, table_hbm, out_hbm,
             idx2, slv, buf, zbuf, isem, zsem, gsem, ssem):
        _body(ids_hbm, len_hbm, table_hbm, out_hbm,
              idx2, slv, buf, zbuf, isem, zsem, gsem, ssem)

    run = pl.kernel(
        body,
        out_type=jax.ShapeDtypeStruct((B, L, D), jnp.float32),
        mesh=plsc.VectorSubcoreMesh(core_axis_name="c", subcore_axis_name="s"),
        compiler_params=pltpu.CompilerParams(needs_layout_passes=False),
        scratch_types=(
            [pltpu.VMEM((NP, P), jnp.int32),
             pltpu.VMEM((16,), jnp.int32),
             pltpu.VMEM((NBUF * P, D), jnp.float32),
             pltpu.VMEM((G, D), jnp.float32),
             pltpu.SemaphoreType.DMA,
             pltpu.SemaphoreType.DMA,
             pltpu.SemaphoreType.DMA((NBUF,)),
             pltpu.SemaphoreType.DMA((NBUF,))]
        ),
    )
    return run(seq_ids, seq_len, table)


# speedup vs baseline: 1.0360x; 1.0360x over previous
import functools

import jax
import jax.numpy as jnp
from jax import lax
from jax.experimental import pallas as pl
from jax.experimental.pallas import tpu as pltpu
from jax.experimental.pallas import tpu_sc as plsc

VOCAB = 30522
D = 768
B = 16
L = 512

NC = 2
NS = 16
NW = NC * NS
TOK = B * L
CH = TOK // NW
PW = L // CH
P = 32
NP = CH // P
NBUF = 4
G = 32
DV = D // 16


def _body(ids_hbm, len_hbm, table_hbm, out_hbm,
          idx2, slv, buf, zbuf, isem, zsem, gsem, ssem):
    wid = lax.axis_index("s") * NC + lax.axis_index("c")
    b = wid // PW
    l_start = (wid % PW) * CH

    def stage_idx(i):
        return pltpu.make_async_copy(
            ids_hbm.at[b, pl.ds(l_start + i * P, P)], idx2.at[i], isem)

    def gather_cp(j, s):
        return pltpu.make_async_copy(
            table_hbm.at[idx2.at[j]],
            buf.at[pl.ds(pl.multiple_of(s * P, P), P)], gsem.at[s])

    def unit_cp(j, s, i, sem):
        return pltpu.make_async_copy(
            buf.at[pl.ds(pl.multiple_of(s * P + i * G, G), G)],
            out_hbm.at[b, pl.ds(pl.multiple_of(l_start + j * P + i * G, G), G)],
            sem)

    stage_idx(0).start()
    stage_idx(1).start()
    stage_idx(0).wait()
    gather_cp(0, 0).start()
    stage_idx(1).wait()
    gather_cp(1, 1).start()
    for i in range(2, NP):
        stage_idx(i).start()

    pltpu.sync_copy(len_hbm, slv)
    for i in range(2, NP):
        stage_idx(i).wait()

    lane = lax.broadcasted_iota(jnp.int32, (16,), 0)
    sl = jnp.max(jnp.where(lane == b, slv[...], 0))
    nv = lax.max(lax.min(sl - l_start, CH), 0)

    zeros16 = jnp.zeros((16,), jnp.float32)

    def zrow(r, _):
        for c in range(DV):
            zbuf[r, pl.ds(c * 16, 16)] = zeros16
        return 0

    lax.fori_loop(0, G, zrow, 0)

    def valid_rows(j):
        return lax.max(lax.min(nv - j * P, P), 0)

    def step(j, ztot):
        s = j % NBUF
        row0 = l_start + j * P
        lo = valid_rows(j)
        a16 = ((lo + (G - 1)) // G) * G
        nu = a16 // G
        nz = (P - a16) // G

        def zfill(i, t):
            pltpu.make_async_copy(
                zbuf,
                out_hbm.at[b, pl.ds(pl.multiple_of(row0 + a16 + i * G, G), G)],
                zsem).start()
            return t + 1

        ztot = lax.fori_loop(0, nz, zfill, ztot)

        def gw(i, _):
            gather_cp(j, s).wait()
            return 0

        lax.fori_loop(0, jnp.where(j < 2, 1, lax.min(nu, 1)), gw, 0)

        def zo(k, _):
            buf[s * P + lo + k // DV, pl.ds((k % DV) * 16, 16)] = zeros16
            return 0

        lax.fori_loop(0, (a16 - lo) * DV, zo, 0)

        def wb(i, _):
            unit_cp(j, s, i, ssem.at[s]).start()
            return 0

        lax.fori_loop(0, nu, wb, 0)

        jj = j + 2
        s2 = jj % NBUF
        live = jj < NP
        jp = j - 2
        lop = valid_rows(jp)
        nup = (lop + (G - 1)) // G
        drain_cnt = jnp.where(live & (jp >= 0), nup, 0)

        def sw(i, _):
            unit_cp(jp, s2, i, ssem.at[s2]).wait()
            return 0

        lax.fori_loop(0, drain_cnt, sw, 0)

        loj = valid_rows(jj)
        fire_cnt = jnp.where(live, lax.min((loj + (G - 1)) // G, 1), 0)

        def gf(i, _):
            gather_cp(jj, s2).start()
            return 0

        lax.fori_loop(0, fire_cnt, gf, 0)
        return ztot

    ztot = lax.fori_loop(0, NP, step, jnp.int32(0))

    def ep(t, _):
        j = (NP - NBUF) + t
        s = j % NBUF
        nu = (valid_rows(j) + (G - 1)) // G

        def sw2(i, _):
            unit_cp(j, s, i, ssem.at[s]).wait()
            return 0

        lax.fori_loop(0, nu, sw2, 0)
        return 0

    lax.fori_loop(0, NBUF, ep, 0)

    def zdrain(i, _):
        pltpu.make_async_copy(
            zbuf, out_hbm.at[b, pl.ds(l_start, G)], zsem).wait()
        return 0

    lax.fori_loop(0, ztot, zdrain, 0)


@functools.partial(jax.jit, static_argnames=())
def kernel(seq_ids, seq_len, table):
    def body(ids_hbm, len_hbm, table_hbm, out_hbm,
             idx2, slv, buf, zbuf, isem, zsem, gsem, ssem):
        _body(ids_hbm, len_hbm, table_hbm, out_hbm,
              idx2, slv, buf, zbuf, isem, zsem, gsem, ssem)

    run = pl.kernel(
        body,
        out_type=jax.ShapeDtypeStruct((B, L, D), jnp.float32),
        mesh=plsc.VectorSubcoreMesh(core_axis_name="c", subcore_axis_name="s"),
        compiler_params=pltpu.CompilerParams(needs_layout_passes=False),
        scratch_types=(
            [pltpu.VMEM((NP, P), jnp.int32),
             pltpu.VMEM((16,), jnp.int32),
             pltpu.VMEM((NBUF * P, D), jnp.float32),
             pltpu.VMEM((G, D), jnp.float32),
             pltpu.SemaphoreType.DMA,
             pltpu.SemaphoreType.DMA,
             pltpu.SemaphoreType.DMA((NBUF,)),
             pltpu.SemaphoreType.DMA((NBUF,))]
        ),
    )
    return run(seq_ids, seq_len, table)


# speedup vs baseline: 1.0434x; 1.0071x over previous
import functools

import jax
import jax.numpy as jnp
from jax import lax
from jax.experimental import pallas as pl
from jax.experimental.pallas import tpu as pltpu
from jax.experimental.pallas import tpu_sc as plsc

VOCAB = 30522
D = 768
B = 16
L = 512

NC = 2
NS = 16
NW = NC * NS
TOK = B * L
CH = TOK // NW
PW = L // CH
P = 64
NP = CH // P
G = 16
DV = D // 16


def _body(ids_hbm, len_hbm, table_hbm, out_hbm,
          idx_refs, slv, bufA, bufB, zbuf, isem, zsem, gsems, ssems):
    wid = lax.axis_index("s") * NC + lax.axis_index("c")
    b = wid // PW
    l_start = (wid % PW) * CH

    bufs = (bufA, bufB)

    def stage_idx(i):
        return pltpu.make_async_copy(
            ids_hbm.at[b, pl.ds(l_start + i * P, P)], idx_refs[i], isem)

    def gather(i, s):
        return pltpu.make_async_copy(
            table_hbm.at[idx_refs[i]], bufs[s], gsems[s])

    stage_idx(0).start()
    stage_idx(1).start()
    stage_idx(0).wait()
    gather(0, 0).start()
    stage_idx(1).wait()
    gather(1, 1).start()
    for i in range(2, NP):
        stage_idx(i).start()

    pltpu.sync_copy(len_hbm, slv)
    for i in range(2, NP):
        stage_idx(i).wait()

    lane = lax.broadcasted_iota(jnp.int32, (16,), 0)
    sl = jnp.max(jnp.where(lane == b, slv[...], 0))
    nv = lax.max(lax.min(sl - l_start, CH), 0)

    zeros16 = jnp.zeros((16,), jnp.float32)

    def zrow(r, _):
        for c in range(DV):
            zbuf[r, pl.ds(c * 16, 16)] = zeros16
        return 0

    lax.fori_loop(0, G, zrow, 0)

    ztot = jnp.int32(0)

    for j in range(NP):
        s = j & 1
        buf = bufs[s]
        row0 = l_start + j * P
        lo = lax.max(lax.min(nv - j * P, P), 0)
        a16 = (lo + (G - 1)) & ~(G - 1)
        nu = a16 // G
        nz = (P - a16) // G

        def zfill(i, t):
            pltpu.make_async_copy(
                zbuf,
                out_hbm.at[b, pl.ds(pl.multiple_of(row0 + a16 + i * G, G), G)],
                zsem
            ).start()
            return t + 1

        ztot = lax.fori_loop(0, nz, zfill, ztot)

        if j < 2:
            gather(j, s).wait()
        else:
            def gwait(i, _):
                gather(j, s).wait()
                return 0

            lax.fori_loop(0, lax.min(nu, 1), gwait, 0)

        def zo(k, _):
            buf[lo + k // DV, pl.ds((k % DV) * 16, 16)] = zeros16
            return 0

        lax.fori_loop(0, (a16 - lo) * DV, zo, 0)

        def wb(i, _):
            pltpu.make_async_copy(
                buf.at[pl.ds(pl.multiple_of(i * G, G), G)],
                out_hbm.at[b, pl.ds(pl.multiple_of(row0 + i * G, G), G)], ssems[s]
            ).start()
            return 0

        lax.fori_loop(0, nu, wb, 0)

        if j + 2 < NP:
            def swait(i, _):
                pltpu.make_async_copy(
                    buf.at[pl.ds(pl.multiple_of(i * G, G), G)],
                    out_hbm.at[b, pl.ds(pl.multiple_of(row0 + i * G, G), G)], ssems[s]
                ).wait()
                return 0

            lax.fori_loop(0, nu, swait, 0)

            lo2 = lax.max(lax.min(nv - (j + 2) * P, P), 0)
            nu2 = lax.min((lo2 + (G - 1)) // G, 1)

            def gfire(i, _):
                gather(j + 2, s).start()
                return 0

            lax.fori_loop(0, nu2, gfire, 0)

    for j in (NP - 2, NP - 1):
        s = j & 1
        row0 = l_start + j * P
        lo = lax.max(lax.min(nv - j * P, P), 0)
        nu = ((lo + (G - 1)) & ~(G - 1)) // G

        def swait2(i, _):
            pltpu.make_async_copy(
                bufs[s].at[pl.ds(pl.multiple_of(i * G, G), G)],
                out_hbm.at[b, pl.ds(pl.multiple_of(row0 + i * G, G), G)], ssems[s]
            ).wait()
            return 0

        lax.fori_loop(0, nu, swait2, 0)

    def zdrain(i, _):
        pltpu.make_async_copy(
            zbuf, out_hbm.at[b, pl.ds(l_start, G)], zsem).wait()
        return 0

    lax.fori_loop(0, ztot, zdrain, 0)


@functools.partial(jax.jit, static_argnames=())
def kernel(seq_ids, seq_len, table):
    def body(ids_hbm, len_hbm, table_hbm, out_hbm, *rest):
        idx_refs = rest[:NP]
        slv = rest[NP]
        bufA, bufB, zbuf = rest[NP + 1:NP + 4]
        isem, zsem = rest[NP + 4:NP + 6]
        gsems = rest[NP + 6:NP + 8]
        ssems = rest[NP + 8:NP + 10]
        _body(ids_hbm, len_hbm, table_hbm, out_hbm,
              idx_refs, slv, bufA, bufB, zbuf, isem, zsem, gsems, ssems)

    run = pl.kernel(
        body,
        out_type=jax.ShapeDtypeStruct((B, L, D), jnp.float32),
        mesh=plsc.VectorSubcoreMesh(core_axis_name="c", subcore_axis_name="s"),
        compiler_params=pltpu.CompilerParams(needs_layout_passes=False),
        scratch_types=(
            [pltpu.VMEM((P,), jnp.int32) for _ in range(NP)]
            + [pltpu.VMEM((16,), jnp.int32)]
            + [pltpu.VMEM((P, D), jnp.float32) for _ in range(2)]
            + [pltpu.VMEM((G, D), jnp.float32)]
            + [pltpu.SemaphoreType.DMA for _ in range(6)]
        ),
    )
    return run(seq_ids, seq_len, table)
